# baseline (speedup 1.0000x reference)
import functools

import jax
import jax.numpy as jnp
from jax import lax
from jax.experimental import pallas as pl
from jax.experimental.pallas import tpu as pltpu
from jax.experimental.pallas import tpu_sc as plsc

N = 10000
NP = 10240
D = 128
E = 320000
NT = 16
STEPS = 160
CH = 40
EPT = STEPS * 128
EPAD = NT * EPT
RPT = NP // NT
TRASH = N
BR = 512

_mesh = plsc.VectorSubcoreMesh(
    core_axis_name="c", subcore_axis_name="s", num_cores=2, num_subcores=NT)


@functools.partial(
    pl.kernel,
    out_type=jax.ShapeDtypeStruct((2 * NP, 16), jnp.float32),
    mesh=_mesh,
    scratch_types=[
        pltpu.VMEM_SHARED((NP, 16), jnp.float32),
        pltpu.VMEM((STEPS, 128), jnp.int32),
        pltpu.VMEM((128, 16), jnp.float32),
        pltpu.VMEM((128, 16), jnp.float32),
    ],
    compiler_params=pltpu.CompilerParams(use_tc_tiling_on_sc=False),
)
def _sc_degree(sidx_hbm, z_hbm, o_hbm, deg_hbm, acc, si, ones_v, zv):
    c = lax.axis_index("c")
    s = lax.axis_index("s")
    r0 = s * RPT
    pltpu.sync_copy(z_hbm, zv)
    pltpu.sync_copy(o_hbm, ones_v)
    pltpu.sync_copy(sidx_hbm.at[c, s], si)
    for k in range(RPT // 128):
        pltpu.sync_copy(zv, acc.at[pl.ds(r0 + k * 128, 128)])
    plsc.subcore_barrier()

    def body(j, carry):
        pltpu.sync_copy(ones_v, acc.at[si.at[j]], add=True)
        return carry

    lax.fori_loop(0, STEPS, body, 0)
    plsc.subcore_barrier()
    for k in range(RPT // 128):
        pltpu.sync_copy(acc.at[pl.ds(r0 + k * 128, 128)], zv)
        pltpu.sync_copy(zv, deg_hbm.at[pl.ds(c * NP + r0 + k * 128, 128)])


@functools.partial(
    pl.kernel,
    out_type=jax.ShapeDtypeStruct((2 * NP, D), jnp.float32),
    mesh=_mesh,
    scratch_types=[
        pltpu.VMEM_SHARED((NP, D), jnp.float32),
        pltpu.VMEM((CH, 128), jnp.int32),
        pltpu.VMEM((CH, 128), jnp.int32),
        pltpu.VMEM((128, D), jnp.float32),
        pltpu.SemaphoreType.DMA,
    ],
    compiler_params=pltpu.CompilerParams(use_tc_tiling_on_sc=False),
)
def _sc_aggregate(g_hbm, gidx_hbm, sidx_hbm, agg_hbm, acc, gi, si, buf, sem):
    c = lax.axis_index("c")
    s = lax.axis_index("s")
    r0 = s * RPT
    for k in range(RPT // 128):
        pltpu.sync_copy(g_hbm.at[pl.ds(c * NP + r0 + k * 128, 128)], buf)
        pltpu.sync_copy(buf, acc.at[pl.ds(r0 + k * 128, 128)])
    plsc.subcore_barrier()

    def body(j, carry):
        pltpu.async_copy(g_hbm.at[gi.at[j]], buf, sem).wait()
        pltpu.sync_copy(buf, acc.at[si.at[j]], add=True)
        return carry

    for ch in range(STEPS // CH):
        pltpu.sync_copy(gidx_hbm.at[c, s, pl.ds(ch * CH, CH)], gi)
        pltpu.sync_copy(sidx_hbm.at[c, s, pl.ds(ch * CH, CH)], si)
        lax.fori_loop(0, CH, body, 0)
    plsc.subcore_barrier()
    for k in range(RPT // 128):
        pltpu.sync_copy(acc.at[pl.ds(r0 + k * 128, 128)], buf)
        pltpu.sync_copy(buf, agg_hbm.at[pl.ds(c * NP + r0 + k * 128, 128)])


def _dinv(deg_ref):
    dvf = lax.rsqrt(deg_ref[0, :, 0:1] + 1.0)
    dvb = lax.rsqrt(deg_ref[1, :, 0:1] + 1.0)
    return dvf, dvb


def _prep_body(x_ref, deg_ref, wf_ref, wb_ref, g_ref):
    dvf, dvb = _dinv(deg_ref)
    h = x_ref[...]
    g_ref[0] = dvf * jnp.dot(h, wf_ref[...], preferred_element_type=jnp.float32)
    g_ref[1] = dvb * jnp.dot(h, wb_ref[...], preferred_element_type=jnp.float32)


def _layer_body(agg_ref, deg_ref, wf_ref, wb_ref, bf_ref, bb_ref, g_ref):
    dvf, dvb = _dinv(deg_ref)
    hf = jnp.maximum(dvf * agg_ref[0] + bf_ref[...], 0.0)
    hb = jnp.maximum(dvb * agg_ref[1] + bb_ref[...], 0.0)
    h = jnp.concatenate([hf, hb], axis=1)
    g_ref[0] = dvf * jnp.dot(h, wf_ref[...], preferred_element_type=jnp.float32)
    g_ref[1] = dvb * jnp.dot(h, wb_ref[...], preferred_element_type=jnp.float32)


def _final_body(agg_ref, deg_ref, bf_ref, bb_ref, o_ref):
    dvf, dvb = _dinv(deg_ref)
    o_ref[:, 0:D] = dvf * agg_ref[0] + bf_ref[...]
    o_ref[:, D : 2 * D] = dvb * agg_ref[1] + bb_ref[...]


_G_SPEC = pl.BlockSpec((2, BR, D), lambda i: (0, i, 0))
_DEG_SPEC = pl.BlockSpec((2, BR, 16), lambda i: (0, i, 0))
_BIAS_SPEC = pl.BlockSpec((1, D), lambda i: (0, 0))


def _tc_prep(x_pad, deg, Wf, Wb):
    return pl.pallas_call(
        _prep_body,
        grid=(NP // BR,),
        in_specs=[
            pl.BlockSpec((BR, D), lambda i: (i, 0)),
            _DEG_SPEC,
            pl.BlockSpec((D, D), lambda i: (0, 0)),
            pl.BlockSpec((D, D), lambda i: (0, 0)),
        ],
        out_specs=_G_SPEC,
        out_shape=jax.ShapeDtypeStruct((2, NP, D), jnp.float32),
    )(x_pad, deg, Wf, Wb)


def _tc_layer(agg, deg, Wf, Wb, bf, bb):
    return pl.pallas_call(
        _layer_body,
        grid=(NP // BR,),
        in_specs=[
            _G_SPEC,
            _DEG_SPEC,
            pl.BlockSpec((2 * D, D), lambda i: (0, 0)),
            pl.BlockSpec((2 * D, D), lambda i: (0, 0)),
            _BIAS_SPEC,
            _BIAS_SPEC,
        ],
        out_specs=_G_SPEC,
        out_shape=jax.ShapeDtypeStruct((2, NP, D), jnp.float32),
    )(agg, deg, Wf, Wb, bf, bb)


def _tc_final(agg, deg, bf, bb):
    return pl.pallas_call(
        _final_body,
        grid=(NP // BR,),
        in_specs=[_G_SPEC, _DEG_SPEC, _BIAS_SPEC, _BIAS_SPEC],
        out_specs=pl.BlockSpec((BR, 2 * D), lambda i: (i, 0)),
        out_shape=jax.ShapeDtypeStruct((NP, 2 * D), jnp.float32),
    )(agg, deg, bf, bb)


def kernel(x, edge_index, Wf0, bf0, Wb0, bb0, Wf1, bf1, Wb1, bb1,
           Wf2, bf2, Wb2, bb2):
    pads = EPAD - E
    fill = jnp.full((pads,), TRASH, jnp.int32)
    src = jnp.concatenate([edge_index[0], fill])
    dst = jnp.concatenate([edge_index[1], fill])
    gidx = jnp.stack([src, dst + NP]).reshape(2, NT, STEPS, 128)
    sidx = jnp.stack([dst, src]).reshape(2, NT, STEPS, 128)

    x_pad = jnp.pad(x, ((0, NP - N), (0, 0)))
    zeros16 = jnp.zeros((128, 16), jnp.float32)
    ones16 = jnp.ones((128, 16), jnp.float32)

    deg = _sc_degree(sidx, zeros16, ones16).reshape(2, NP, 16)

    g = _tc_prep(x_pad, deg, Wf0, Wb0)
    bf_prev, bb_prev = bf0.reshape(1, D), bb0.reshape(1, D)
    for Wf, bf, Wb, bb in ((Wf1, bf1, Wb1, bb1), (Wf2, bf2, Wb2, bb2)):
        agg = _sc_aggregate(g.reshape(2 * NP, D), gidx, sidx).reshape(2, NP, D)
        g = _tc_layer(agg, deg, Wf, Wb, bf_prev, bb_prev)
        bf_prev, bb_prev = bf.reshape(1, D), bb.reshape(1, D)
    agg = _sc_aggregate(g.reshape(2 * NP, D), gidx, sidx).reshape(2, NP, D)
    out = _tc_final(agg, deg, bf_prev, bb_prev)
    return out[:N]


# speedup vs baseline: 2.0736x; 2.0736x over previous
import functools

import jax
import jax.numpy as jnp
from jax import lax
from jax.experimental import pallas as pl
from jax.experimental.pallas import tpu as pltpu
from jax.experimental.pallas import tpu_sc as plsc

N = 10000
NP = 10240
D = 128
E = 320000
NT = 16
STEPS = 160
CH = 40
EPT = STEPS * 128
EPAD = NT * EPT
RPT = NP // NT
TRASH = N
BR = 512

_mesh = plsc.VectorSubcoreMesh(
    core_axis_name="c", subcore_axis_name="s", num_cores=2, num_subcores=NT)


@functools.partial(
    pl.kernel,
    out_type=jax.ShapeDtypeStruct((2 * NP, 16), jnp.float32),
    mesh=_mesh,
    scratch_types=[
        pltpu.VMEM_SHARED((NP, 16), jnp.float32),
        pltpu.VMEM((STEPS, 128), jnp.int32),
        pltpu.VMEM((128, 16), jnp.float32),
        pltpu.VMEM((128, 16), jnp.float32),
    ],
    compiler_params=pltpu.CompilerParams(use_tc_tiling_on_sc=False),
)
def _sc_degree(sidx_hbm, z_hbm, o_hbm, deg_hbm, acc, si, ones_v, zv):
    c = lax.axis_index("c")
    s = lax.axis_index("s")
    r0 = s * RPT
    pltpu.sync_copy(z_hbm, zv)
    pltpu.sync_copy(o_hbm, ones_v)
    pltpu.sync_copy(sidx_hbm.at[c, s], si)
    for k in range(RPT // 128):
        pltpu.sync_copy(zv, acc.at[pl.ds(r0 + k * 128, 128)])
    plsc.subcore_barrier()

    def body(j, carry):
        pltpu.sync_copy(ones_v, acc.at[si.at[j]], add=True)
        return carry

    lax.fori_loop(0, STEPS, body, 0)
    plsc.subcore_barrier()
    for k in range(RPT // 128):
        pltpu.sync_copy(acc.at[pl.ds(r0 + k * 128, 128)], zv)
        pltpu.sync_copy(zv, deg_hbm.at[pl.ds(c * NP + r0 + k * 128, 128)])


@functools.partial(
    pl.kernel,
    out_type=jax.ShapeDtypeStruct((2 * NP, D), jnp.float32),
    mesh=_mesh,
    scratch_types=[
        pltpu.VMEM_SHARED((NP, D), jnp.float32),
        pltpu.VMEM((CH, 128), jnp.int32),
        pltpu.VMEM((CH, 128), jnp.int32),
        pltpu.VMEM((128, D), jnp.float32),
        pltpu.SemaphoreType.DMA,
    ],
    compiler_params=pltpu.CompilerParams(use_tc_tiling_on_sc=False),
)
def _sc_aggregate(g_hbm, gidx_hbm, sidx_hbm, agg_hbm, acc, gi, si, buf, sem):
    c = lax.axis_index("c")
    s = lax.axis_index("s")
    r0 = s * RPT
    for k in range(RPT // 128):
        pltpu.sync_copy(g_hbm.at[pl.ds(c * NP + r0 + k * 128, 128)], buf)
        pltpu.sync_copy(buf, acc.at[pl.ds(r0 + k * 128, 128)])
    plsc.subcore_barrier()

    def body(j, carry):
        pltpu.async_copy(g_hbm.at[gi.at[j]], buf, sem).wait()
        pltpu.sync_copy(buf, acc.at[si.at[j]], add=True)
        return carry

    for ch in range(STEPS // CH):
        pltpu.sync_copy(gidx_hbm.at[c, s, pl.ds(ch * CH, CH)], gi)
        pltpu.sync_copy(sidx_hbm.at[c, s, pl.ds(ch * CH, CH)], si)
        lax.fori_loop(0, CH, body, 0)
    plsc.subcore_barrier()
    for k in range(RPT // 128):
        pltpu.sync_copy(acc.at[pl.ds(r0 + k * 128, 128)], buf)
        pltpu.sync_copy(buf, agg_hbm.at[pl.ds(c * NP + r0 + k * 128, 128)])


def _dinv(deg_ref):
    dvf = lax.rsqrt(deg_ref[0, :, 0:1] + 1.0)
    dvb = lax.rsqrt(deg_ref[1, :, 0:1] + 1.0)
    return dvf, dvb


def _prep_body(x_ref, deg_ref, wf_ref, wb_ref, g_ref):
    dvf, dvb = _dinv(deg_ref)
    h = x_ref[...]
    g_ref[0] = dvf * jnp.dot(h, wf_ref[...], preferred_element_type=jnp.float32)
    g_ref[1] = dvb * jnp.dot(h, wb_ref[...], preferred_element_type=jnp.float32)


def _layer_body(agg_ref, deg_ref, wf_ref, wb_ref, bf_ref, bb_ref, g_ref):
    dvf, dvb = _dinv(deg_ref)
    hf = jnp.maximum(dvf * agg_ref[0] + bf_ref[...], 0.0)
    hb = jnp.maximum(dvb * agg_ref[1] + bb_ref[...], 0.0)
    h = jnp.concatenate([hf, hb], axis=1)
    g_ref[0] = dvf * jnp.dot(h, wf_ref[...], preferred_element_type=jnp.float32)
    g_ref[1] = dvb * jnp.dot(h, wb_ref[...], preferred_element_type=jnp.float32)


def _final_body(agg_ref, deg_ref, bf_ref, bb_ref, o_ref):
    dvf, dvb = _dinv(deg_ref)
    o_ref[:, 0:D] = dvf * agg_ref[0] + bf_ref[...]
    o_ref[:, D : 2 * D] = dvb * agg_ref[1] + bb_ref[...]


_G_SPEC = pl.BlockSpec((2, BR, D), lambda i: (0, i, 0))
_DEG_SPEC = pl.BlockSpec((2, BR, 16), lambda i: (0, i, 0))
_BIAS_SPEC = pl.BlockSpec((1, D), lambda i: (0, 0))


def _tc_prep(x_pad, deg, Wf, Wb):
    return pl.pallas_call(
        _prep_body,
        grid=(NP // BR,),
        in_specs=[
            pl.BlockSpec((BR, D), lambda i: (i, 0)),
            _DEG_SPEC,
            pl.BlockSpec((D, D), lambda i: (0, 0)),
            pl.BlockSpec((D, D), lambda i: (0, 0)),
        ],
        out_specs=_G_SPEC,
        out_shape=jax.ShapeDtypeStruct((2, NP, D), jnp.float32),
    )(x_pad, deg, Wf, Wb)


def _tc_layer(agg, deg, Wf, Wb, bf, bb):
    return pl.pallas_call(
        _layer_body,
        grid=(NP // BR,),
        in_specs=[
            _G_SPEC,
            _DEG_SPEC,
            pl.BlockSpec((2 * D, D), lambda i: (0, 0)),
            pl.BlockSpec((2 * D, D), lambda i: (0, 0)),
            _BIAS_SPEC,
            _BIAS_SPEC,
        ],
        out_specs=_G_SPEC,
        out_shape=jax.ShapeDtypeStruct((2, NP, D), jnp.float32),
    )(agg, deg, Wf, Wb, bf, bb)


def _tc_final(agg, deg, bf, bb):
    return pl.pallas_call(
        _final_body,
        grid=(NP // BR,),
        in_specs=[_G_SPEC, _DEG_SPEC, _BIAS_SPEC, _BIAS_SPEC],
        out_specs=pl.BlockSpec((BR, 2 * D), lambda i: (i, 0)),
        out_shape=jax.ShapeDtypeStruct((NP, 2 * D), jnp.float32),
    )(agg, deg, bf, bb)


def kernel(x, edge_index, Wf0, bf0, Wb0, bb0, Wf1, bf1, Wb1, bb1,
           Wf2, bf2, Wb2, bb2):
    pads = EPAD - E
    fill = N + (jnp.arange(pads, dtype=jnp.int32) % (NP - N))
    src = jnp.concatenate([edge_index[0], fill])
    dst = jnp.concatenate([edge_index[1], fill])
    gidx = jnp.stack([src, dst + NP]).reshape(2, NT, STEPS, 128)
    sidx = jnp.stack([dst, src]).reshape(2, NT, STEPS, 128)

    x_pad = jnp.pad(x, ((0, NP - N), (0, 0)))
    zeros16 = jnp.zeros((128, 16), jnp.float32)
    ones16 = jnp.ones((128, 16), jnp.float32)

    deg = _sc_degree(sidx, zeros16, ones16).reshape(2, NP, 16)

    g = _tc_prep(x_pad, deg, Wf0, Wb0)
    bf_prev, bb_prev = bf0.reshape(1, D), bb0.reshape(1, D)
    for Wf, bf, Wb, bb in ((Wf1, bf1, Wb1, bb1), (Wf2, bf2, Wb2, bb2)):
        agg = _sc_aggregate(g.reshape(2 * NP, D), gidx, sidx).reshape(2, NP, D)
        g = _tc_layer(agg, deg, Wf, Wb, bf_prev, bb_prev)
        bf_prev, bb_prev = bf.reshape(1, D), bb.reshape(1, D)
    agg = _sc_aggregate(g.reshape(2 * NP, D), gidx, sidx).reshape(2, NP, D)
    out = _tc_final(agg, deg, bf_prev, bb_prev)
    return out[:N]
